# Initial kernel scaffold; baseline (speedup 1.0000x reference)
import jax
import jax.numpy as jnp
from jax.experimental import pallas as pl


def kernel(x, edge_index, W1, b1, W2, b2):
    raise NotImplementedError("write your pallas kernel here")


# speedup vs baseline: 42.8198x; 42.8198x over previous
import functools

import jax
import jax.numpy as jnp
from jax import lax
from jax.experimental import pallas as pl
from jax.experimental.pallas import tpu as pltpu
from jax.experimental.pallas import tpu_sc as plsc

N = 10000
NPAD = 10240
E = 320000
NC, NS, LANES = 2, 16, 16
NT = NC * NS
B = 128
K = 80
EPT = K * B
EPAD = NT * EPT
RPT = NPAD // NS

_mesh = plsc.VectorSubcoreMesh(
    core_axis_name="c", subcore_axis_name="s", num_cores=NC, num_subcores=NS)


@functools.partial(
    pl.kernel,
    out_type=jax.ShapeDtypeStruct((NC, NS, RPT), jnp.float32),
    mesh=_mesh,
    scratch_types=[
        pltpu.VMEM((K, B), jnp.int32),
        pltpu.VMEM((B,), jnp.float32),
        pltpu.VMEM((RPT,), jnp.float32),
        pltpu.VMEM_SHARED((NPAD,), jnp.float32),
    ],
    compiler_params=pltpu.CompilerParams(use_tc_tiling_on_sc=False),
)
def _deg_kernel(dstT_hbm, out_hbm, dstv, onesv, zbuf, acc):
    c = lax.axis_index("c")
    s = lax.axis_index("s")
    wid = s * NC + c
    one = jnp.ones((LANES,), jnp.float32)
    zero = jnp.zeros((LANES,), jnp.float32)
    for i in range(0, B, LANES):
        onesv[pl.ds(i, LANES)] = one
    for i in range(0, RPT, LANES):
        zbuf[pl.ds(i, LANES)] = zero
    pltpu.sync_copy(zbuf, acc.at[pl.ds(s * RPT, RPT)])
    pltpu.sync_copy(dstT_hbm.at[wid], dstv)
    plsc.subcore_barrier()

    def step(j, carry):
        pltpu.sync_copy(onesv, acc.at[dstv.at[j]], add=True)
        return carry

    lax.fori_loop(0, K, step, 0)
    plsc.subcore_barrier()
    pltpu.sync_copy(acc.at[pl.ds(s * RPT, RPT)], out_hbm.at[c, s])


def _make_agg(D):
    @functools.partial(
        pl.kernel,
        out_type=jax.ShapeDtypeStruct((NC, NS, RPT, D), jnp.float32),
        mesh=_mesh,
        scratch_types=[
            pltpu.VMEM((K, B), jnp.int32),
            pltpu.VMEM((K, B), jnp.int32),
            pltpu.VMEM((B, D), jnp.float32),
            pltpu.VMEM((B, D), jnp.float32),
            pltpu.VMEM_SHARED((NPAD, D), jnp.float32),
            pltpu.SemaphoreType.DMA,
            pltpu.SemaphoreType.DMA,
        ],
        compiler_params=pltpu.CompilerParams(use_tc_tiling_on_sc=False),
    )
    def agg(hs_hbm, srcT_hbm, dstT_hbm, zer_hbm, out_hbm,
            srcv, dstv, buf0, buf1, acc, sem0, sem1):
        c = lax.axis_index("c")
        s = lax.axis_index("s")
        wid = s * NC + c
        pltpu.sync_copy(zer_hbm.at[pl.ds(s * RPT, RPT)],
                        acc.at[pl.ds(s * RPT, RPT)])
        pltpu.sync_copy(srcT_hbm.at[wid], srcv)
        pltpu.sync_copy(dstT_hbm.at[wid], dstv)
        plsc.subcore_barrier()

        pltpu.async_copy(hs_hbm.at[srcv.at[0]], buf0, sem0)
        pltpu.async_copy(hs_hbm.at[srcv.at[1]], buf1, sem1)

        def pair(j, carry):
            pltpu.make_async_copy(hs_hbm.at[srcv.at[j]], buf0, sem0).wait()
            pltpu.sync_copy(buf0, acc.at[dstv.at[j]], add=True)

            @pl.when(j + 2 < K)
            def _():
                pltpu.async_copy(hs_hbm.at[srcv.at[j + 2]], buf0, sem0)

            pltpu.make_async_copy(hs_hbm.at[srcv.at[j + 1]], buf1, sem1).wait()
            pltpu.sync_copy(buf1, acc.at[dstv.at[j + 1]], add=True)

            @pl.when(j + 3 < K)
            def _():
                pltpu.async_copy(hs_hbm.at[srcv.at[j + 3]], buf1, sem1)

            return carry

        lax.fori_loop(0, K // 2, lambda i, cr: pair(2 * i, cr), 0)
        plsc.subcore_barrier()
        pltpu.sync_copy(acc.at[pl.ds(s * RPT, RPT)], out_hbm.at[c, s])

    return agg


_agg32 = _make_agg(32)
_agg8 = _make_agg(8)


def _k2_body(x_ref, w1_ref, degt_ref, hs_ref, dinv_ref):
    deg = jnp.sum(degt_ref[...], axis=1, keepdims=True) + 1.0
    dinv = lax.rsqrt(deg)
    dinv_ref[...] = dinv
    h1 = jnp.dot(x_ref[...], w1_ref[...], preferred_element_type=jnp.float32)
    hs_ref[...] = h1 * dinv


def _k4_body(pa_ref, pb_ref, hs1_ref, dinv_ref, b1_ref, w2_ref, hs2_ref):
    dinv = dinv_ref[...]
    full = pa_ref[...] + pb_ref[...] + hs1_ref[...]
    h = jnp.maximum(full * dinv + b1_ref[...], 0.0)
    h2 = jnp.dot(h, w2_ref[...], preferred_element_type=jnp.float32)
    hs2_ref[...] = h2 * dinv


def _k6_body(pa_ref, pb_ref, hs2_ref, dinv_ref, b2_ref, out_ref):
    logits = ((pa_ref[...] + pb_ref[...] + hs2_ref[...]) * dinv_ref[...]
              + b2_ref[...])
    m = jnp.max(logits, axis=1, keepdims=True)
    e = jnp.exp(logits - m)
    out_ref[...] = (e / jnp.sum(e, axis=1, keepdims=True))[:, :7]


@jax.jit
def kernel(x, edge_index, W1, b1, W2, b2):
    f32 = jnp.float32
    src = edge_index[0].astype(jnp.int32)
    dst = edge_index[1].astype(jnp.int32)
    npad_e = EPAD - E
    pad_i = jnp.arange(npad_e, dtype=jnp.int32)
    srcT = jnp.concatenate([src, pad_i % N]).reshape(NT, K, B)
    dstT = jnp.concatenate([dst, N + pad_i % (NPAD - N)]).reshape(NT, K, B)

    xp = jnp.concatenate([x, jnp.zeros((NPAD - N, x.shape[1]), f32)])
    w2p = jnp.concatenate([W2, jnp.zeros((W2.shape[0], 1), f32)], axis=1)
    b1r = b1.reshape(1, -1)
    b2r = jnp.concatenate([b2, jnp.full((1,), -1e30, f32)]).reshape(1, 8)
    zer32 = jnp.zeros((NPAD, 32), f32)
    zer8 = jnp.zeros((NPAD, 8), f32)

    degp = _deg_kernel(dstT)
    degt = degp.transpose(1, 2, 0).reshape(NPAD, NC)

    hs1, dinv = pl.pallas_call(
        _k2_body,
        out_shape=[jax.ShapeDtypeStruct((NPAD, 32), f32),
                   jax.ShapeDtypeStruct((NPAD, 1), f32)],
    )(xp, W1, degt)

    p1 = _agg32(hs1, srcT, dstT, zer32)
    hs2 = pl.pallas_call(
        _k4_body,
        out_shape=jax.ShapeDtypeStruct((NPAD, 8), f32),
    )(p1[0].reshape(NPAD, 32), p1[1].reshape(NPAD, 32), hs1, dinv, b1r, w2p)

    p2 = _agg8(hs2, srcT, dstT, zer8)
    out = pl.pallas_call(
        _k6_body,
        out_shape=jax.ShapeDtypeStruct((NPAD, 7), f32),
    )(p2[0].reshape(NPAD, 8), p2[1].reshape(NPAD, 8), hs2, dinv, b2r)
    return out[:N]


# speedup vs baseline: 45.1408x; 1.0542x over previous
import functools

import jax
import jax.numpy as jnp
from jax import lax
from jax.experimental import pallas as pl
from jax.experimental.pallas import tpu as pltpu
from jax.experimental.pallas import tpu_sc as plsc

N = 10000
NPAD = 10240
E = 320000
NC, NS, LANES = 2, 16, 16
NT = NC * NS
B = 128
K = 80
EPT = K * B
EPAD = NT * EPT
RPT = NPAD // NS

_mesh = plsc.VectorSubcoreMesh(
    core_axis_name="c", subcore_axis_name="s", num_cores=NC, num_subcores=NS)


@functools.partial(
    pl.kernel,
    out_type=jax.ShapeDtypeStruct((NC, NS, RPT), jnp.float32),
    mesh=_mesh,
    scratch_types=[
        pltpu.VMEM((K, B), jnp.int32),
        pltpu.VMEM((B,), jnp.float32),
        pltpu.VMEM((RPT,), jnp.float32),
        pltpu.VMEM_SHARED((NPAD,), jnp.float32),
        pltpu.SemaphoreType.DMA,
    ],
    compiler_params=pltpu.CompilerParams(use_tc_tiling_on_sc=False),
)
def _deg_kernel(dstT_hbm, out_hbm, dstv, onesv, zbuf, acc, sem):
    c = lax.axis_index("c")
    s = lax.axis_index("s")
    wid = s * NC + c
    one = jnp.ones((LANES,), jnp.float32)
    zero = jnp.zeros((LANES,), jnp.float32)
    for i in range(0, B, LANES):
        onesv[pl.ds(i, LANES)] = one
    for i in range(0, RPT, LANES):
        zbuf[pl.ds(i, LANES)] = zero
    pltpu.sync_copy(zbuf, acc.at[pl.ds(s * RPT, RPT)])
    pltpu.sync_copy(dstT_hbm.at[wid], dstv)
    plsc.subcore_barrier()

    def fire(j, carry):
        pltpu.async_copy(onesv, acc.at[dstv.at[j]], sem, add=True)
        return carry

    lax.fori_loop(0, K, fire, 0)

    def drain(j, carry):
        pltpu.make_async_copy(onesv, acc.at[dstv.at[j]], sem).wait()
        return carry

    lax.fori_loop(0, K, drain, 0)
    plsc.subcore_barrier()
    pltpu.sync_copy(acc.at[pl.ds(s * RPT, RPT)], out_hbm.at[c, s])


def _make_agg(D):
    @functools.partial(
        pl.kernel,
        out_type=jax.ShapeDtypeStruct((NC, NS, RPT, D), jnp.float32),
        mesh=_mesh,
        scratch_types=[
            pltpu.VMEM((K, B), jnp.int32),
            pltpu.VMEM((K, B), jnp.int32),
            pltpu.VMEM((B, D), jnp.float32),
            pltpu.VMEM((B, D), jnp.float32),
            pltpu.VMEM((B, D), jnp.float32),
            pltpu.VMEM((B, D), jnp.float32),
            pltpu.VMEM_SHARED((NPAD, D), jnp.float32),
            pltpu.SemaphoreType.DMA,
            pltpu.SemaphoreType.DMA,
            pltpu.SemaphoreType.DMA,
            pltpu.SemaphoreType.DMA,
            pltpu.SemaphoreType.DMA,
            pltpu.SemaphoreType.DMA,
            pltpu.SemaphoreType.DMA,
            pltpu.SemaphoreType.DMA,
        ],
        compiler_params=pltpu.CompilerParams(use_tc_tiling_on_sc=False),
    )
    def agg(hs_hbm, srcT_hbm, dstT_hbm, zer_hbm, out_hbm,
            srcv, dstv, b0, b1, b2, b3, acc,
            g0, g1, g2, g3, s0, s1, s2, s3):
        bufs = (b0, b1, b2, b3)
        gsem = (g0, g1, g2, g3)
        ssem = (s0, s1, s2, s3)
        c = lax.axis_index("c")
        s = lax.axis_index("s")
        wid = s * NC + c
        pltpu.sync_copy(zer_hbm.at[pl.ds(s * RPT, RPT)],
                        acc.at[pl.ds(s * RPT, RPT)])
        pltpu.sync_copy(srcT_hbm.at[wid], srcv)
        pltpu.sync_copy(dstT_hbm.at[wid], dstv)
        plsc.subcore_barrier()

        pltpu.async_copy(hs_hbm.at[srcv.at[0]], bufs[0], gsem[0])
        pltpu.async_copy(hs_hbm.at[srcv.at[1]], bufs[1], gsem[1])

        def round4(jo, carry):
            for r in range(4):
                idx = 4 * jo + r
                rn = (r + 2) % 4
                pltpu.make_async_copy(
                    hs_hbm.at[srcv.at[idx]], bufs[r], gsem[r]).wait()
                pltpu.async_copy(
                    bufs[r], acc.at[dstv.at[idx]], ssem[r], add=True)

                @pl.when(idx >= 2)
                def _():
                    pltpu.make_async_copy(
                        bufs[rn], acc.at[dstv.at[idx - 2]], ssem[rn]).wait()

                @pl.when(idx + 2 < K)
                def _():
                    pltpu.async_copy(
                        hs_hbm.at[srcv.at[idx + 2]], bufs[rn], gsem[rn])
            return carry

        lax.fori_loop(0, K // 4, round4, 0)
        pltpu.make_async_copy(bufs[(K - 2) % 4],
                              acc.at[dstv.at[K - 2]], ssem[(K - 2) % 4]).wait()
        pltpu.make_async_copy(bufs[(K - 1) % 4],
                              acc.at[dstv.at[K - 1]], ssem[(K - 1) % 4]).wait()
        plsc.subcore_barrier()
        pltpu.sync_copy(acc.at[pl.ds(s * RPT, RPT)], out_hbm.at[c, s])

    return agg


_agg32 = _make_agg(32)
_agg8 = _make_agg(8)


def _k2a_body(x_ref, w1_ref, h1_ref):
    h1_ref[...] = jnp.dot(x_ref[...], w1_ref[...],
                          preferred_element_type=jnp.float32)


def _k2b_body(h1_ref, degt_ref, hs_ref, dinv_ref):
    deg = jnp.sum(degt_ref[...], axis=1, keepdims=True) + 1.0
    dinv = lax.rsqrt(deg)
    dinv_ref[...] = dinv
    hs_ref[...] = h1_ref[...] * dinv


def _k4_body(pa_ref, pb_ref, hs1_ref, dinv_ref, b1_ref, w2_ref, hs2_ref):
    dinv = dinv_ref[...]
    full = pa_ref[...] + pb_ref[...] + hs1_ref[...]
    h = jnp.maximum(full * dinv + b1_ref[...], 0.0)
    h2 = jnp.dot(h, w2_ref[...], preferred_element_type=jnp.float32)
    hs2_ref[...] = h2 * dinv


def _k6_body(pa_ref, pb_ref, hs2_ref, dinv_ref, b2_ref, out_ref):
    logits = ((pa_ref[...] + pb_ref[...] + hs2_ref[...]) * dinv_ref[...]
              + b2_ref[...])
    m = jnp.max(logits, axis=1, keepdims=True)
    e = jnp.exp(logits - m)
    out_ref[...] = (e / jnp.sum(e, axis=1, keepdims=True))[:, :7]


@jax.jit
def kernel(x, edge_index, W1, b1, W2, b2):
    f32 = jnp.float32
    src = edge_index[0].astype(jnp.int32)
    dst = edge_index[1].astype(jnp.int32)
    npad_e = EPAD - E
    pad_i = jnp.arange(npad_e, dtype=jnp.int32)
    srcT = jnp.concatenate([src, pad_i % N]).reshape(NT, K, B)
    dstT = jnp.concatenate([dst, N + pad_i % (NPAD - N)]).reshape(NT, K, B)

    xp = jnp.concatenate([x, jnp.zeros((NPAD - N, x.shape[1]), f32)])
    w2p = jnp.concatenate([W2, jnp.zeros((W2.shape[0], 1), f32)], axis=1)
    b1r = b1.reshape(1, -1)
    b2r = jnp.concatenate([b2, jnp.full((1,), -1e30, f32)]).reshape(1, 8)
    zer32 = jnp.zeros((NPAD, 32), f32)
    zer8 = jnp.zeros((NPAD, 8), f32)

    degp = _deg_kernel(dstT)
    degt = degp.transpose(1, 2, 0).reshape(NPAD, NC)

    h1 = pl.pallas_call(
        _k2a_body,
        out_shape=jax.ShapeDtypeStruct((NPAD, 32), f32),
    )(xp, W1)
    hs1, dinv = pl.pallas_call(
        _k2b_body,
        out_shape=[jax.ShapeDtypeStruct((NPAD, 32), f32),
                   jax.ShapeDtypeStruct((NPAD, 1), f32)],
    )(h1, degt)

    p1 = _agg32(hs1, srcT, dstT, zer32)
    hs2 = pl.pallas_call(
        _k4_body,
        out_shape=jax.ShapeDtypeStruct((NPAD, 8), f32),
    )(p1[0].reshape(NPAD, 32), p1[1].reshape(NPAD, 32), hs1, dinv, b1r, w2p)

    p2 = _agg8(hs2, srcT, dstT, zer8)
    out = pl.pallas_call(
        _k6_body,
        out_shape=jax.ShapeDtypeStruct((NPAD, 7), f32),
    )(p2[0].reshape(NPAD, 8), p2[1].reshape(NPAD, 8), hs2, dinv, b2r)
    return out[:N]


# speedup vs baseline: 53.4769x; 1.1847x over previous
import functools

import jax
import jax.numpy as jnp
from jax import lax
from jax.experimental import pallas as pl
from jax.experimental.pallas import tpu as pltpu
from jax.experimental.pallas import tpu_sc as plsc

N = 10000
NPAD = 10240
E = 320000
NC, NS, LANES = 2, 16, 16
NT = NC * NS
B = 128
K = 80
EPT = K * B
EPAD = NT * EPT
RPT = NPAD // NS

_mesh = plsc.VectorSubcoreMesh(
    core_axis_name="c", subcore_axis_name="s", num_cores=NC, num_subcores=NS)


@functools.partial(
    pl.kernel,
    out_type=jax.ShapeDtypeStruct((NC, NS, RPT), jnp.float32),
    mesh=_mesh,
    scratch_types=[
        pltpu.VMEM((K, B), jnp.int32),
        pltpu.VMEM((B,), jnp.float32),
        pltpu.VMEM((RPT,), jnp.float32),
        pltpu.VMEM_SHARED((NPAD,), jnp.float32),
        pltpu.SemaphoreType.DMA,
    ],
    compiler_params=pltpu.CompilerParams(use_tc_tiling_on_sc=False),
)
def _deg_kernel(dstT_hbm, out_hbm, dstv, onesv, zbuf, acc, sem):
    c = lax.axis_index("c")
    s = lax.axis_index("s")
    wid = s * NC + c
    one = jnp.ones((LANES,), jnp.float32)
    zero = jnp.zeros((LANES,), jnp.float32)
    for i in range(0, B, LANES):
        onesv[pl.ds(i, LANES)] = one
    for i in range(0, RPT, LANES):
        zbuf[pl.ds(i, LANES)] = zero
    pltpu.sync_copy(zbuf, acc.at[pl.ds(s * RPT, RPT)])
    pltpu.sync_copy(dstT_hbm.at[wid], dstv)
    plsc.subcore_barrier()

    def fire(j, carry):
        pltpu.async_copy(onesv, acc.at[dstv.at[j]], sem, add=True)
        return carry

    lax.fori_loop(0, K, fire, 0)

    def drain(j, carry):
        pltpu.make_async_copy(onesv, acc.at[dstv.at[j]], sem).wait()
        return carry

    lax.fori_loop(0, K, drain, 0)
    plsc.subcore_barrier()
    pltpu.sync_copy(acc.at[pl.ds(s * RPT, RPT)], out_hbm.at[c, s])


NB = 8
GL = 6


def _make_agg(D):
    @functools.partial(
        pl.kernel,
        out_type=jax.ShapeDtypeStruct((NC, NS, RPT, D), jnp.float32),
        mesh=_mesh,
        scratch_types=[
            pltpu.VMEM((K, B), jnp.int32),
            pltpu.VMEM((K, B), jnp.int32),
            [pltpu.VMEM((B, D), jnp.float32)] * 8,
            pltpu.VMEM_SHARED((NPAD, D), jnp.float32),
            [pltpu.SemaphoreType.DMA] * 8,
            [pltpu.SemaphoreType.DMA] * 8,
        ],
        compiler_params=pltpu.CompilerParams(use_tc_tiling_on_sc=False),
    )
    def agg(hs_hbm, srcT_hbm, dstT_hbm, zer_hbm, out_hbm,
            srcv, dstv, bufs, acc, gsem, ssem):
        c = lax.axis_index("c")
        s = lax.axis_index("s")
        wid = s * NC + c
        pltpu.sync_copy(zer_hbm.at[pl.ds(s * RPT, RPT)],
                        acc.at[pl.ds(s * RPT, RPT)])
        pltpu.sync_copy(srcT_hbm.at[wid], srcv)
        pltpu.sync_copy(dstT_hbm.at[wid], dstv)
        plsc.subcore_barrier()

        for p in range(GL):
            pltpu.async_copy(hs_hbm.at[srcv.at[p]], bufs[p], gsem[p])

        def roundn(jo, carry):
            for r in range(NB):
                idx = NB * jo + r
                rn = (r + GL) % NB
                pltpu.make_async_copy(
                    hs_hbm.at[srcv.at[idx]], bufs[r], gsem[r]).wait()
                pltpu.async_copy(
                    bufs[r], acc.at[dstv.at[idx]], ssem[r], add=True)

                @pl.when(idx >= NB - GL)
                def _():
                    pltpu.make_async_copy(
                        bufs[rn], acc.at[dstv.at[idx - (NB - GL)]],
                        ssem[rn]).wait()

                @pl.when(idx + GL < K)
                def _():
                    pltpu.async_copy(
                        hs_hbm.at[srcv.at[idx + GL]], bufs[rn], gsem[rn])
            return carry

        lax.fori_loop(0, K // NB, roundn, 0)
        for t in range(K - (NB - GL), K):
            pltpu.make_async_copy(bufs[t % NB],
                                  acc.at[dstv.at[t]], ssem[t % NB]).wait()
        plsc.subcore_barrier()
        pltpu.sync_copy(acc.at[pl.ds(s * RPT, RPT)], out_hbm.at[c, s])

    return agg


_agg32 = _make_agg(32)
_agg8 = _make_agg(8)


def _k2a_body(x_ref, w1_ref, h1_ref):
    h1_ref[...] = jnp.dot(x_ref[...], w1_ref[...],
                          preferred_element_type=jnp.float32)


def _k2b_body(h1_ref, degt_ref, hs_ref, dinv_ref):
    deg = jnp.sum(degt_ref[...], axis=1, keepdims=True) + 1.0
    dinv = lax.rsqrt(deg)
    dinv_ref[...] = dinv
    hs_ref[...] = h1_ref[...] * dinv


def _k4_body(pa_ref, pb_ref, hs1_ref, dinv_ref, b1_ref, w2_ref, hs2_ref):
    dinv = dinv_ref[...]
    full = pa_ref[...] + pb_ref[...] + hs1_ref[...]
    h = jnp.maximum(full * dinv + b1_ref[...], 0.0)
    h2 = jnp.dot(h, w2_ref[...], preferred_element_type=jnp.float32)
    hs2_ref[...] = h2 * dinv


def _k6_body(pa_ref, pb_ref, hs2_ref, dinv_ref, b2_ref, out_ref):
    logits = ((pa_ref[...] + pb_ref[...] + hs2_ref[...]) * dinv_ref[...]
              + b2_ref[...])
    m = jnp.max(logits, axis=1, keepdims=True)
    e = jnp.exp(logits - m)
    out_ref[...] = (e / jnp.sum(e, axis=1, keepdims=True))[:, :7]


@jax.jit
def kernel(x, edge_index, W1, b1, W2, b2):
    f32 = jnp.float32
    src = edge_index[0].astype(jnp.int32)
    dst = edge_index[1].astype(jnp.int32)
    npad_e = EPAD - E
    pad_i = jnp.arange(npad_e, dtype=jnp.int32)
    srcT = jnp.concatenate([src, pad_i % N]).reshape(NT, K, B)
    dstT = jnp.concatenate([dst, N + pad_i % (NPAD - N)]).reshape(NT, K, B)

    xp = jnp.concatenate([x, jnp.zeros((NPAD - N, x.shape[1]), f32)])
    w2p = jnp.concatenate([W2, jnp.zeros((W2.shape[0], 1), f32)], axis=1)
    b1r = b1.reshape(1, -1)
    b2r = jnp.concatenate([b2, jnp.full((1,), -1e30, f32)]).reshape(1, 8)
    zer32 = jnp.zeros((NPAD, 32), f32)
    zer8 = jnp.zeros((NPAD, 8), f32)

    degp = _deg_kernel(dstT)
    degt = degp.transpose(1, 2, 0).reshape(NPAD, NC)

    h1 = pl.pallas_call(
        _k2a_body,
        out_shape=jax.ShapeDtypeStruct((NPAD, 32), f32),
    )(xp, W1)
    hs1, dinv = pl.pallas_call(
        _k2b_body,
        out_shape=[jax.ShapeDtypeStruct((NPAD, 32), f32),
                   jax.ShapeDtypeStruct((NPAD, 1), f32)],
    )(h1, degt)

    p1 = _agg32(hs1, srcT, dstT, zer32)
    hs2 = pl.pallas_call(
        _k4_body,
        out_shape=jax.ShapeDtypeStruct((NPAD, 8), f32),
    )(p1[0].reshape(NPAD, 32), p1[1].reshape(NPAD, 32), hs1, dinv, b1r, w2p)

    p2 = _agg8(hs2, srcT, dstT, zer8)
    out = pl.pallas_call(
        _k6_body,
        out_shape=jax.ShapeDtypeStruct((NPAD, 7), f32),
    )(p2[0].reshape(NPAD, 8), p2[1].reshape(NPAD, 8), hs2, dinv, b2r)
    return out[:N]
